# Initial kernel scaffold; baseline (speedup 1.0000x reference)
import jax
import jax.numpy as jnp
from jax.experimental import pallas as pl


def kernel(input_ids, pos_table):
    raise NotImplementedError("write your pallas kernel here")


# speedup vs baseline: 2.7517x; 2.7517x over previous
import jax
import jax.numpy as jnp
from jax.experimental import pallas as pl

_BLOCK_ROWS = 512


def _lookup_rows_kernel(tbl_ref, out_ref):
    out_ref[...] = tbl_ref[...]


def kernel(input_ids, pos_table):
    seq_len = input_ids.shape[1]
    hidden = pos_table.shape[1]
    block = min(_BLOCK_ROWS, seq_len)
    out = pl.pallas_call(
        _lookup_rows_kernel,
        out_shape=jax.ShapeDtypeStruct((seq_len, hidden), pos_table.dtype),
        grid=(pl.cdiv(seq_len, block),),
        in_specs=[pl.BlockSpec((block, hidden), lambda i: (i, 0))],
        out_specs=pl.BlockSpec((block, hidden), lambda i: (i, 0)),
    )(pos_table)
    return out[None]
